# baseline (device time: 101625 ns/iter reference)
import functools

import jax
import jax.numpy as jnp
from jax import lax
from jax.experimental import pallas as pl
from jax.experimental.pallas import tpu as pltpu

N_DEV = 8


def kernel(t, W):
    m, k = t.shape
    _, n = W.shape

    def body(t_ref, w_ref, out_ref, acc_ref, comm_ref, send_sems, recv_sems):
        my = lax.axis_index("i")
        left = (my + N_DEV - 1) % N_DEV
        right = (my + 1) % N_DEV

        barrier_sem = pltpu.get_barrier_semaphore()
        for nbr in (left, right):
            pl.semaphore_signal(
                barrier_sem, inc=1,
                device_id=(nbr,), device_id_type=pl.DeviceIdType.MESH,
            )
        pl.semaphore_wait(barrier_sem, 2)

        acc_ref[...] = t_ref[...]
        comm_ref[0] = t_ref[...].astype(jnp.bfloat16)

        for h in range(N_DEV - 1):
            rdma = pltpu.make_async_remote_copy(
                src_ref=comm_ref.at[h],
                dst_ref=comm_ref.at[h + 1],
                send_sem=send_sems.at[h],
                recv_sem=recv_sems.at[h],
                device_id=(right,),
                device_id_type=pl.DeviceIdType.MESH,
            )
            rdma.start()
            rdma.wait()
            acc_ref[...] += comm_ref[h + 1].astype(jnp.float32)

        out_ref[...] = jax.lax.dot(
            acc_ref[...].astype(jnp.bfloat16),
            w_ref[...].astype(jnp.bfloat16),
            preferred_element_type=jnp.float32,
        )

        @functools.partial(pl.run_scoped, exit_sem=pltpu.SemaphoreType.REGULAR)
        def _(exit_sem):
            for nbr in (left, right):
                pl.semaphore_signal(
                    exit_sem, inc=1,
                    device_id=(nbr,), device_id_type=pl.DeviceIdType.MESH,
                )
            pl.semaphore_wait(exit_sem, 2)

    return pl.pallas_call(
        body,
        out_shape=jax.ShapeDtypeStruct((m, n), jnp.float32),
        in_specs=[
            pl.BlockSpec(memory_space=pltpu.VMEM),
            pl.BlockSpec(memory_space=pltpu.VMEM),
        ],
        out_specs=pl.BlockSpec(memory_space=pltpu.VMEM),
        scratch_shapes=[
            pltpu.VMEM((m, k), jnp.float32),
            pltpu.VMEM((N_DEV, m, k), jnp.bfloat16),
            pltpu.SemaphoreType.DMA((N_DEV - 1,)),
            pltpu.SemaphoreType.DMA((N_DEV - 1,)),
        ],
        compiler_params=pltpu.CompilerParams(collective_id=0),
    )(t, W)


# device time: 37977 ns/iter; 2.6760x vs baseline; 2.6760x over previous
import jax
import jax.numpy as jnp
from jax import lax
from jax.experimental import pallas as pl
from jax.experimental.pallas import tpu as pltpu

N_DEV = 8


def kernel(t, W):
    m, k = t.shape
    _, n = W.shape
    H, Q, E = m // 2, m // 4, m // 8

    def body(t_ref, w_ref, out_ref, acc_ref, rb_h, rb_q, rb_e,
             send_sems, recv_sems):
        my = lax.axis_index("i")
        px = my ^ 1
        py = my ^ 3
        pz = my ^ 4
        xb = (my ^ (my >> 1)) & 1
        yb = (my >> 1) & 1
        zb = (my >> 2) & 1

        barrier_sem = pltpu.get_barrier_semaphore()
        for nbr in (px, py, pz):
            pl.semaphore_signal(
                barrier_sem, inc=1,
                device_id=(nbr,), device_id_type=pl.DeviceIdType.MESH,
            )
        pl.semaphore_wait(barrier_sem, 3)

        acc_ref[...] = t_ref[...].astype(jnp.bfloat16)

        off_h = xb * H
        off_q = off_h + yb * Q
        off_e = off_q + zb * E

        rs_rounds = [
            (px, (1 - xb) * H, off_h, H, rb_h, 0),
            (py, off_h + (1 - yb) * Q, off_q, Q, rb_q, 1),
            (pz, off_q + (1 - zb) * E, off_e, E, rb_e, 2),
        ]
        for p, soff, koff, sz, rb, r in rs_rounds:
            rdma = pltpu.make_async_remote_copy(
                src_ref=acc_ref.at[pl.ds(soff, sz)],
                dst_ref=rb,
                send_sem=send_sems.at[r],
                recv_sem=recv_sems.at[r],
                device_id=(p,),
                device_id_type=pl.DeviceIdType.MESH,
            )
            rdma.start()
            rdma.wait()
            acc_ref[pl.ds(koff, sz)] += rb[...]

        ag_rounds = [(pz, off_e, E, 3), (py, off_q, Q, 4), (px, off_h, H, 5)]
        for p, off, sz, r in ag_rounds:
            rdma = pltpu.make_async_remote_copy(
                src_ref=acc_ref.at[pl.ds(off, sz)],
                dst_ref=acc_ref.at[pl.ds(off, sz)],
                send_sem=send_sems.at[r],
                recv_sem=recv_sems.at[r],
                device_id=(p,),
                device_id_type=pl.DeviceIdType.MESH,
            )
            rdma.start()
            rdma.wait()

        out_ref[...] = jax.lax.dot(
            acc_ref[...],
            w_ref[...].astype(jnp.bfloat16),
            preferred_element_type=jnp.float32,
        )

    return pl.pallas_call(
        body,
        out_shape=jax.ShapeDtypeStruct((m, n), jnp.float32),
        in_specs=[
            pl.BlockSpec(memory_space=pltpu.VMEM),
            pl.BlockSpec(memory_space=pltpu.VMEM),
        ],
        out_specs=pl.BlockSpec(memory_space=pltpu.VMEM),
        scratch_shapes=[
            pltpu.VMEM((m, k), jnp.bfloat16),
            pltpu.VMEM((H, k), jnp.bfloat16),
            pltpu.VMEM((Q, k), jnp.bfloat16),
            pltpu.VMEM((E, k), jnp.bfloat16),
            pltpu.SemaphoreType.DMA((6,)),
            pltpu.SemaphoreType.DMA((6,)),
        ],
        compiler_params=pltpu.CompilerParams(collective_id=0),
    )(t, W)


# device time: 27132 ns/iter; 3.7456x vs baseline; 1.3997x over previous
import jax
import jax.numpy as jnp
from jax import lax
from jax.experimental import pallas as pl
from jax.experimental.pallas import tpu as pltpu

N_DEV = 8


def kernel(t, W):
    m, k = t.shape
    _, n = W.shape
    E = m // N_DEV

    def body(t_ref, w_ref, out_ref, acc_ref, red_ref, rbuf,
             rs_send_sems, rs_recv_sems, ag_send_sems, ag_recv_sems):
        my = lax.axis_index("i")
        off_e = my * E

        barrier_sem = pltpu.get_barrier_semaphore()
        for j in range(1, N_DEV):
            pl.semaphore_signal(
                barrier_sem, inc=1,
                device_id=(my ^ j,), device_id_type=pl.DeviceIdType.MESH,
            )
        pl.semaphore_wait(barrier_sem, N_DEV - 1)

        acc_ref[...] = t_ref[...].astype(jnp.bfloat16)

        rs = []
        for j in range(1, N_DEV):
            p = my ^ j
            rdma = pltpu.make_async_remote_copy(
                src_ref=acc_ref.at[pl.ds(p * E, E)],
                dst_ref=rbuf.at[j - 1],
                send_sem=rs_send_sems.at[j - 1],
                recv_sem=rs_recv_sems.at[j - 1],
                device_id=(p,),
                device_id_type=pl.DeviceIdType.MESH,
            )
            rdma.start()
            rs.append(rdma)

        red_ref[...] = t_ref[pl.ds(off_e, E)]
        for j in range(1, N_DEV):
            rs[j - 1].wait()
            red_ref[...] += rbuf[j - 1].astype(jnp.float32)
        acc_ref[pl.ds(off_e, E)] = red_ref[...].astype(jnp.bfloat16)

        ag = []
        for j in range(1, N_DEV):
            p = my ^ j
            rdma = pltpu.make_async_remote_copy(
                src_ref=acc_ref.at[pl.ds(off_e, E)],
                dst_ref=acc_ref.at[pl.ds(off_e, E)],
                send_sem=ag_send_sems.at[j - 1],
                recv_sem=ag_recv_sems.at[j - 1],
                device_id=(p,),
                device_id_type=pl.DeviceIdType.MESH,
            )
            rdma.start()
            ag.append(rdma)
        for r in ag:
            r.wait()

        out_ref[...] = jax.lax.dot(
            acc_ref[...],
            w_ref[...].astype(jnp.bfloat16),
            preferred_element_type=jnp.float32,
        )

    return pl.pallas_call(
        body,
        out_shape=jax.ShapeDtypeStruct((m, n), jnp.float32),
        in_specs=[
            pl.BlockSpec(memory_space=pltpu.VMEM),
            pl.BlockSpec(memory_space=pltpu.VMEM),
        ],
        out_specs=pl.BlockSpec(memory_space=pltpu.VMEM),
        scratch_shapes=[
            pltpu.VMEM((m, k), jnp.bfloat16),
            pltpu.VMEM((E, k), jnp.float32),
            pltpu.VMEM((N_DEV - 1, E, k), jnp.bfloat16),
            pltpu.SemaphoreType.DMA((N_DEV - 1,)),
            pltpu.SemaphoreType.DMA((N_DEV - 1,)),
            pltpu.SemaphoreType.DMA((N_DEV - 1,)),
            pltpu.SemaphoreType.DMA((N_DEV - 1,)),
        ],
        compiler_params=pltpu.CompilerParams(collective_id=0),
    )(t, W)


# device time: 24317 ns/iter; 4.1792x vs baseline; 1.1158x over previous
import jax
import jax.numpy as jnp
from jax import lax
from jax.experimental import pallas as pl
from jax.experimental.pallas import tpu as pltpu

N_DEV = 8


def kernel(t, W):
    m, k = t.shape
    _, n = W.shape
    E = m // N_DEV

    def body(t_ref, w_ref, out_ref, acc_ref, wbf_ref, red_ref, rbuf,
             rs_send_sems, rs_recv_sems, ag_send_sems, ag_recv_sems):
        my = lax.axis_index("i")
        off_e = my * E

        barrier_sem = pltpu.get_barrier_semaphore()
        for j in range(1, N_DEV):
            pl.semaphore_signal(
                barrier_sem, inc=1,
                device_id=(my ^ j,), device_id_type=pl.DeviceIdType.MESH,
            )
        pl.semaphore_wait(barrier_sem, N_DEV - 1)

        rs = []
        for j in range(1, N_DEV):
            p = my ^ j
            acc_ref[pl.ds(p * E, E)] = t_ref[pl.ds(p * E, E)].astype(jnp.bfloat16)
            rdma = pltpu.make_async_remote_copy(
                src_ref=acc_ref.at[pl.ds(p * E, E)],
                dst_ref=rbuf.at[j - 1],
                send_sem=rs_send_sems.at[j - 1],
                recv_sem=rs_recv_sems.at[j - 1],
                device_id=(p,),
                device_id_type=pl.DeviceIdType.MESH,
            )
            rdma.start()
            rs.append(rdma)

        wbf_ref[...] = w_ref[...].astype(jnp.bfloat16)
        red_ref[...] = t_ref[pl.ds(off_e, E)]
        for j in range(1, N_DEV):
            rs[j - 1].wait()
            red_ref[...] += rbuf[j - 1].astype(jnp.float32)
        acc_ref[pl.ds(off_e, E)] = red_ref[...].astype(jnp.bfloat16)

        ag = []
        for j in range(1, N_DEV):
            p = my ^ j
            rdma = pltpu.make_async_remote_copy(
                src_ref=acc_ref.at[pl.ds(off_e, E)],
                dst_ref=acc_ref.at[pl.ds(off_e, E)],
                send_sem=ag_send_sems.at[j - 1],
                recv_sem=ag_recv_sems.at[j - 1],
                device_id=(p,),
                device_id_type=pl.DeviceIdType.MESH,
            )
            rdma.start()
            ag.append(rdma)

        out_ref[pl.ds(off_e, E)] = jax.lax.dot(
            acc_ref[pl.ds(off_e, E)], wbf_ref[...],
            preferred_element_type=jnp.float32,
        )
        for j in range(1, N_DEV):
            p = my ^ j
            ag[j - 1].wait()
            out_ref[pl.ds(p * E, E)] = jax.lax.dot(
                acc_ref[pl.ds(p * E, E)], wbf_ref[...],
                preferred_element_type=jnp.float32,
            )

    return pl.pallas_call(
        body,
        out_shape=jax.ShapeDtypeStruct((m, n), jnp.float32),
        in_specs=[
            pl.BlockSpec(memory_space=pltpu.VMEM),
            pl.BlockSpec(memory_space=pltpu.VMEM),
        ],
        out_specs=pl.BlockSpec(memory_space=pltpu.VMEM),
        scratch_shapes=[
            pltpu.VMEM((m, k), jnp.bfloat16),
            pltpu.VMEM((k, n), jnp.bfloat16),
            pltpu.VMEM((E, k), jnp.float32),
            pltpu.VMEM((N_DEV - 1, E, k), jnp.bfloat16),
            pltpu.SemaphoreType.DMA((N_DEV - 1,)),
            pltpu.SemaphoreType.DMA((N_DEV - 1,)),
            pltpu.SemaphoreType.DMA((N_DEV - 1,)),
            pltpu.SemaphoreType.DMA((N_DEV - 1,)),
        ],
        compiler_params=pltpu.CompilerParams(collective_id=0),
    )(t, W)


# device time: 23804 ns/iter; 4.2692x vs baseline; 1.0216x over previous
import jax
import jax.numpy as jnp
from jax import lax
from jax.experimental import pallas as pl
from jax.experimental.pallas import tpu as pltpu

N_DEV = 8
S = 2


def kernel(t, W):
    m, k = t.shape
    _, n = W.shape
    E = m // N_DEV
    U = E // S

    def body(t_ref, w_ref, out_ref, acc_ref, wbf_ref, red_ref, rbuf,
             rs_send_sems, rs_recv_sems, ag_send_sems, ag_recv_sems):
        my = lax.axis_index("i")
        off_e = my * E

        barrier_sem = pltpu.get_barrier_semaphore()
        for j in range(1, N_DEV):
            pl.semaphore_signal(
                barrier_sem, inc=1,
                device_id=(my ^ j,), device_id_type=pl.DeviceIdType.MESH,
            )
        pl.semaphore_wait(barrier_sem, N_DEV - 1)

        rs = {}
        for s in range(S):
            for j in range(1, N_DEV):
                p = my ^ j
                src = pl.ds(p * E + s * U, U)
                acc_ref[src] = t_ref[src].astype(jnp.bfloat16)
                idx = (j - 1) * S + s
                rdma = pltpu.make_async_remote_copy(
                    src_ref=acc_ref.at[src],
                    dst_ref=rbuf.at[idx],
                    send_sem=rs_send_sems.at[idx],
                    recv_sem=rs_recv_sems.at[idx],
                    device_id=(p,),
                    device_id_type=pl.DeviceIdType.MESH,
                )
                rdma.start()
                rs[idx] = rdma

        wbf_ref[...] = w_ref[...].astype(jnp.bfloat16)

        ag = {}
        for s in range(S):
            sl = pl.ds(off_e + s * U, U)
            red_ref[pl.ds(s * U, U)] = t_ref[sl]
            for j in range(1, N_DEV):
                idx = (j - 1) * S + s
                rs[idx].wait()
                red_ref[pl.ds(s * U, U)] += rbuf[idx].astype(jnp.float32)
            acc_ref[sl] = red_ref[pl.ds(s * U, U)].astype(jnp.bfloat16)
            for j in range(1, N_DEV):
                p = my ^ j
                idx = (j - 1) * S + s
                rdma = pltpu.make_async_remote_copy(
                    src_ref=acc_ref.at[sl],
                    dst_ref=acc_ref.at[sl],
                    send_sem=ag_send_sems.at[idx],
                    recv_sem=ag_recv_sems.at[idx],
                    device_id=(p,),
                    device_id_type=pl.DeviceIdType.MESH,
                )
                rdma.start()
                ag[idx] = rdma

        out_ref[pl.ds(off_e, E)] = jax.lax.dot(
            acc_ref[pl.ds(off_e, E)], wbf_ref[...],
            preferred_element_type=jnp.float32,
        )
        for j in range(1, N_DEV):
            p = my ^ j
            for s in range(S):
                ag[(j - 1) * S + s].wait()
            out_ref[pl.ds(p * E, E)] = jax.lax.dot(
                acc_ref[pl.ds(p * E, E)], wbf_ref[...],
                preferred_element_type=jnp.float32,
            )

    return pl.pallas_call(
        body,
        out_shape=jax.ShapeDtypeStruct((m, n), jnp.float32),
        in_specs=[
            pl.BlockSpec(memory_space=pltpu.VMEM),
            pl.BlockSpec(memory_space=pltpu.VMEM),
        ],
        out_specs=pl.BlockSpec(memory_space=pltpu.VMEM),
        scratch_shapes=[
            pltpu.VMEM((m, k), jnp.bfloat16),
            pltpu.VMEM((k, n), jnp.bfloat16),
            pltpu.VMEM((E, k), jnp.float32),
            pltpu.VMEM(((N_DEV - 1) * S, U, k), jnp.bfloat16),
            pltpu.SemaphoreType.DMA(((N_DEV - 1) * S,)),
            pltpu.SemaphoreType.DMA(((N_DEV - 1) * S,)),
            pltpu.SemaphoreType.DMA(((N_DEV - 1) * S,)),
            pltpu.SemaphoreType.DMA(((N_DEV - 1) * S,)),
        ],
        compiler_params=pltpu.CompilerParams(collective_id=0),
    )(t, W)


# device time: 22437 ns/iter; 4.5293x vs baseline; 1.0609x over previous
import jax
import jax.numpy as jnp
from jax import lax
from jax.experimental import pallas as pl
from jax.experimental.pallas import tpu as pltpu

N_DEV = 8
S = 2

J_ORDER = (1, 3, 4, 2, 5, 7, 6)


def kernel(t, W):
    m, k = t.shape
    _, n = W.shape
    E = m // N_DEV
    U = E // S

    def body(t_ref, w_ref, out_ref, acc_ref, wbf_ref, red_ref, rbuf,
             ready_sems, rs_send_sems, rs_recv_sems, ag_send_sems,
             ag_recv_sems):
        my = lax.axis_index("i")
        off_e = my * E

        barrier_sem = pltpu.get_barrier_semaphore()
        pl.semaphore_signal(barrier_sem, inc=1)
        pl.semaphore_wait(barrier_sem, 1)

        for j in range(1, N_DEV):
            pl.semaphore_signal(
                ready_sems.at[j - 1], inc=1,
                device_id=(my ^ j,), device_id_type=pl.DeviceIdType.MESH,
            )

        rs = {}
        for s in range(S):
            for j in J_ORDER:
                p = my ^ j
                if s == 0:
                    pl.semaphore_wait(ready_sems.at[j - 1], 1)
                src = pl.ds(p * E + s * U, U)
                acc_ref[src] = t_ref[src].astype(jnp.bfloat16)
                idx = (j - 1) * S + s
                rdma = pltpu.make_async_remote_copy(
                    src_ref=acc_ref.at[src],
                    dst_ref=rbuf.at[idx],
                    send_sem=rs_send_sems.at[idx],
                    recv_sem=rs_recv_sems.at[idx],
                    device_id=(p,),
                    device_id_type=pl.DeviceIdType.MESH,
                )
                rdma.start()
                rs[idx] = rdma

        wbf_ref[...] = w_ref[...].astype(jnp.bfloat16)

        ag = {}
        for s in range(S):
            sl = pl.ds(off_e + s * U, U)
            red_ref[pl.ds(s * U, U)] = t_ref[sl]
            for j in J_ORDER:
                idx = (j - 1) * S + s
                rs[idx].wait()
                red_ref[pl.ds(s * U, U)] += rbuf[idx].astype(jnp.float32)
            acc_ref[sl] = red_ref[pl.ds(s * U, U)].astype(jnp.bfloat16)
            for j in J_ORDER:
                p = my ^ j
                idx = (j - 1) * S + s
                rdma = pltpu.make_async_remote_copy(
                    src_ref=acc_ref.at[sl],
                    dst_ref=acc_ref.at[sl],
                    send_sem=ag_send_sems.at[idx],
                    recv_sem=ag_recv_sems.at[idx],
                    device_id=(p,),
                    device_id_type=pl.DeviceIdType.MESH,
                )
                rdma.start()
                ag[idx] = rdma

        out_ref[pl.ds(off_e, E)] = jax.lax.dot(
            acc_ref[pl.ds(off_e, E)], wbf_ref[...],
            preferred_element_type=jnp.float32,
        )
        for j in J_ORDER:
            p = my ^ j
            for s in range(S):
                ag[(j - 1) * S + s].wait()
            out_ref[pl.ds(p * E, E)] = jax.lax.dot(
                acc_ref[pl.ds(p * E, E)], wbf_ref[...],
                preferred_element_type=jnp.float32,
            )

    return pl.pallas_call(
        body,
        out_shape=jax.ShapeDtypeStruct((m, n), jnp.float32),
        in_specs=[
            pl.BlockSpec(memory_space=pltpu.VMEM),
            pl.BlockSpec(memory_space=pltpu.VMEM),
        ],
        out_specs=pl.BlockSpec(memory_space=pltpu.VMEM),
        scratch_shapes=[
            pltpu.VMEM((m, k), jnp.bfloat16),
            pltpu.VMEM((k, n), jnp.bfloat16),
            pltpu.VMEM((E, k), jnp.float32),
            pltpu.VMEM(((N_DEV - 1) * S, U, k), jnp.bfloat16),
            pltpu.SemaphoreType.REGULAR((N_DEV - 1,)),
            pltpu.SemaphoreType.DMA(((N_DEV - 1) * S,)),
            pltpu.SemaphoreType.DMA(((N_DEV - 1) * S,)),
            pltpu.SemaphoreType.DMA(((N_DEV - 1) * S,)),
            pltpu.SemaphoreType.DMA(((N_DEV - 1) * S,)),
        ],
        compiler_params=pltpu.CompilerParams(collective_id=0),
    )(t, W)
